# baseline (speedup 1.0000x reference)
import functools

import jax
import jax.numpy as jnp
from jax import lax
from jax.experimental import pallas as pl
from jax.experimental.pallas import tpu as pltpu
from jax.experimental.pallas import tpu_sc as plsc

_NC = 2
_NS = 16
_NW = _NC * _NS
_L = 16
_BBLK = 128


@jax.jit
def _embedding_lookup(tokens_t, token_table, position_embedding):
    T, B = tokens_t.shape
    V, D = token_table.shape
    CB = D // 8
    NB = B // _BBLK
    NG = _BBLK // _L
    KP = T // 2

    mesh = plsc.VectorSubcoreMesh(core_axis_name="c", subcore_axis_name="s")

    @functools.partial(
        pl.kernel,
        mesh=mesh,
        compiler_params=pltpu.CompilerParams(
            use_tc_tiling_on_sc=False, needs_layout_passes=False),
        out_type=jax.ShapeDtypeStruct((T, CB, NB, 8, _BBLK), jnp.float32),
        scratch_types=[
            pltpu.VMEM((T, _BBLK), jnp.int32),
            pltpu.VMEM((_BBLK, D), jnp.float32),
            pltpu.VMEM((_BBLK, D), jnp.float32),
            pltpu.VMEM((CB, 8, _BBLK), jnp.float32),
            pltpu.VMEM((CB, 8, _BBLK), jnp.float32),
            pltpu.VMEM((T, D), jnp.float32),
            pltpu.SemaphoreType.DMA,
            pltpu.SemaphoreType.DMA,
            pltpu.SemaphoreType.DMA,
            pltpu.SemaphoreType.DMA,
        ],
    )
    def emb_kernel(tok_hbm, tab_hbm, pos_hbm, out_hbm,
                   idx_v, gbuf0, gbuf1, obuf0, obuf1, pos_v,
                   g_sem0, g_sem1, w_sem0, w_sem1):
        w = lax.axis_index("s") * _NC + lax.axis_index("c")
        pltpu.sync_copy(tok_hbm.at[:, pl.ds(w * _BBLK, _BBLK)], idx_v)
        pltpu.sync_copy(pos_hbm, pos_v)

        lane = lax.iota(jnp.int32, _L)
        row_ids = [lane + bg * _L for bg in range(NG)]

        def start_gather(t, gbuf, gsem):
            pltpu.async_copy(tab_hbm.at[idx_v.at[t]], gbuf, gsem)

        def wait_gather(t, gbuf, gsem):
            pltpu.make_async_copy(tab_hbm.at[idx_v.at[t]], gbuf, gsem).wait()

        def start_write(t, obuf, wsem):
            pltpu.async_copy(obuf, out_hbm.at[t, :, w, :, :], wsem)

        def wait_write(t, obuf, wsem):
            pltpu.make_async_copy(obuf, out_hbm.at[t, :, w, :, :], wsem).wait()

        def transpose_add(t, gbuf, obuf):
            t_splat = jnp.full((_L,), 0, dtype=jnp.int32) + t

            @plsc.parallel_loop(0, D, unroll=8)
            def _transpose_c(c):
                c_splat = jnp.full((_L,), 0, dtype=jnp.int32) + c
                p = plsc.load_gather(pos_v, [t_splat, c_splat])
                cb = lax.shift_right_logical(c, 3)
                ci = lax.bitwise_and(c, 7)
                for bg in range(NG):
                    v = plsc.load_gather(gbuf, [row_ids[bg], c_splat])
                    obuf[cb, ci, pl.ds(bg * _L, _L)] = v + p

        start_gather(0, gbuf0, g_sem0)
        start_gather(1, gbuf1, g_sem1)

        def pair_body(k, carry):
            t0 = 2 * k
            t1 = t0 + 1

            wait_gather(t0, gbuf0, g_sem0)

            @pl.when(k > 0)
            def _():
                wait_write(t0 - 2, obuf0, w_sem0)

            transpose_add(t0, gbuf0, obuf0)
            start_write(t0, obuf0, w_sem0)

            @pl.when(k < KP - 1)
            def _():
                start_gather(t0 + 2, gbuf0, g_sem0)

            wait_gather(t1, gbuf1, g_sem1)

            @pl.when(k > 0)
            def _():
                wait_write(t1 - 2, obuf1, w_sem1)

            transpose_add(t1, gbuf1, obuf1)
            start_write(t1, obuf1, w_sem1)

            @pl.when(k < KP - 1)
            def _():
                start_gather(t1 + 2, gbuf1, g_sem1)

            return carry

        lax.fori_loop(0, KP, pair_body, 0)

        wait_write(T - 2, obuf0, w_sem0)
        wait_write(T - 1, obuf1, w_sem1)

    return emb_kernel(tokens_t, token_table, position_embedding)


def kernel(tokens, token_table, position_embedding):
    B, T = tokens.shape
    D = token_table.shape[1]
    tokens_t = tokens.T.astype(jnp.int32)
    out5 = _embedding_lookup(tokens_t, token_table, position_embedding)
    return out5.transpose(2, 4, 0, 1, 3).reshape(B, T, D)


# speedup vs baseline: 1.9931x; 1.9931x over previous
import functools

import jax
import jax.numpy as jnp
from jax import lax
from jax.experimental import pallas as pl
from jax.experimental.pallas import tpu as pltpu
from jax.experimental.pallas import tpu_sc as plsc

_NC = 2
_NS = 16
_NW = _NC * _NS
_L = 16
_BBLK = 128


@jax.jit
def _embedding_lookup(tokens_t, token_table, position_embedding):
    T, B = tokens_t.shape
    V, D = token_table.shape
    CB = D // 8
    NB = B // _BBLK
    NG = _BBLK // _L
    KP = T // 2

    mesh = plsc.VectorSubcoreMesh(core_axis_name="c", subcore_axis_name="s")

    @functools.partial(
        pl.kernel,
        mesh=mesh,
        compiler_params=pltpu.CompilerParams(
            use_tc_tiling_on_sc=False, needs_layout_passes=False),
        out_type=jax.ShapeDtypeStruct((T, CB, NB, 8, _BBLK), jnp.float32),
        scratch_types=[
            pltpu.VMEM((T, _BBLK), jnp.int32),
            pltpu.VMEM((_BBLK, D), jnp.float32),
            pltpu.VMEM((_BBLK, D), jnp.float32),
            pltpu.VMEM((CB, 8, _BBLK), jnp.float32),
            pltpu.VMEM((CB, 8, _BBLK), jnp.float32),
            pltpu.VMEM((T, D), jnp.float32),
            pltpu.SemaphoreType.DMA,
            pltpu.SemaphoreType.DMA,
            pltpu.SemaphoreType.DMA,
            pltpu.SemaphoreType.DMA,
        ],
    )
    def emb_kernel(tok_hbm, tab_hbm, pos_hbm, out_hbm,
                   idx_v, gbuf0, gbuf1, obuf0, obuf1, pos_v,
                   g_sem0, g_sem1, w_sem0, w_sem1):
        w = lax.axis_index("s") * _NC + lax.axis_index("c")
        pltpu.sync_copy(tok_hbm.at[:, pl.ds(w * _BBLK, _BBLK)], idx_v)
        pltpu.sync_copy(pos_hbm, pos_v)

        lane = lax.iota(jnp.int32, _L)
        row_ids = [lane + bg * _L for bg in range(NG)]

        def start_gather(t, gbuf, gsem):
            pltpu.async_copy(tab_hbm.at[idx_v.at[t]], gbuf, gsem)

        def wait_gather(t, gbuf, gsem):
            pltpu.make_async_copy(tab_hbm.at[idx_v.at[t]], gbuf, gsem).wait()

        def start_write(t, obuf, wsem):
            pltpu.async_copy(obuf, out_hbm.at[t, :, w, :, :], wsem)

        def wait_write(t, obuf, wsem):
            pltpu.make_async_copy(obuf, out_hbm.at[t, :, w, :, :], wsem).wait()

        def transpose_add(t, gbuf, obuf):
            t_splat = jnp.full((_L,), 0, dtype=jnp.int32) + t

            @plsc.parallel_loop(0, _L, unroll=2)
            def _diag_s(sft):
                rot = lax.bitwise_and(lane + sft, _L - 1)
                for q in range(D // _L):
                    col_ids = rot + q * _L
                    p = plsc.load_gather(pos_v, [t_splat, col_ids])
                    cb_ids = lax.shift_right_logical(col_ids, 3)
                    ci_ids = lax.bitwise_and(col_ids, 7)
                    for bg in range(NG):
                        v = plsc.load_gather(gbuf, [row_ids[bg], col_ids])
                        plsc.store_scatter(
                            obuf, [cb_ids, ci_ids, row_ids[bg]], v + p)

        start_gather(0, gbuf0, g_sem0)
        start_gather(1, gbuf1, g_sem1)

        def pair_body(k, carry):
            t0 = 2 * k
            t1 = t0 + 1

            wait_gather(t0, gbuf0, g_sem0)

            @pl.when(k > 0)
            def _():
                wait_write(t0 - 2, obuf0, w_sem0)

            transpose_add(t0, gbuf0, obuf0)
            start_write(t0, obuf0, w_sem0)

            @pl.when(k < KP - 1)
            def _():
                start_gather(t0 + 2, gbuf0, g_sem0)

            wait_gather(t1, gbuf1, g_sem1)

            @pl.when(k > 0)
            def _():
                wait_write(t1 - 2, obuf1, w_sem1)

            transpose_add(t1, gbuf1, obuf1)
            start_write(t1, obuf1, w_sem1)

            @pl.when(k < KP - 1)
            def _():
                start_gather(t1 + 2, gbuf1, g_sem1)

            return carry

        lax.fori_loop(0, KP, pair_body, 0)

        wait_write(T - 2, obuf0, w_sem0)
        wait_write(T - 1, obuf1, w_sem1)

    return emb_kernel(tokens_t, token_table, position_embedding)


def kernel(tokens, token_table, position_embedding):
    B, T = tokens.shape
    D = token_table.shape[1]
    tokens_t = tokens.T.astype(jnp.int32)
    out5 = _embedding_lookup(tokens_t, token_table, position_embedding)
    return out5.transpose(2, 4, 0, 1, 3).reshape(B, T, D)


# speedup vs baseline: 1.9933x; 1.0001x over previous
import functools

import jax
import jax.numpy as jnp
from jax import lax
from jax.experimental import pallas as pl
from jax.experimental.pallas import tpu as pltpu
from jax.experimental.pallas import tpu_sc as plsc

_NC = 2
_NS = 16
_NW = _NC * _NS
_L = 16
_BBLK = 128


@functools.partial(jax.jit, static_argnums=(3,))
def _embedding_lookup(tokens4, token_table, position_embedding, T):
    S7, NB4, S8, BL = tokens4.shape
    B = NB4 * BL
    V, D = token_table.shape
    CB = D // 8
    NB = B // _BBLK
    NG = _BBLK // _L
    KP = T // 2

    mesh = plsc.VectorSubcoreMesh(core_axis_name="c", subcore_axis_name="s")

    @functools.partial(
        pl.kernel,
        mesh=mesh,
        compiler_params=pltpu.CompilerParams(
            use_tc_tiling_on_sc=False, needs_layout_passes=False),
        out_type=jax.ShapeDtypeStruct((T, CB, NB, 8, _BBLK), jnp.float32),
        scratch_types=[
            pltpu.VMEM((S7, S8, _BBLK), jnp.int32),
            pltpu.VMEM((_BBLK, D), jnp.float32),
            pltpu.VMEM((_BBLK, D), jnp.float32),
            pltpu.VMEM((CB, 8, _BBLK), jnp.float32),
            pltpu.VMEM((CB, 8, _BBLK), jnp.float32),
            pltpu.VMEM((T, D), jnp.float32),
            pltpu.SemaphoreType.DMA,
            pltpu.SemaphoreType.DMA,
            pltpu.SemaphoreType.DMA,
            pltpu.SemaphoreType.DMA,
        ],
    )
    def emb_kernel(tok_hbm, tab_hbm, pos_hbm, out_hbm,
                   idx_v, gbuf0, gbuf1, obuf0, obuf1, pos_v,
                   g_sem0, g_sem1, w_sem0, w_sem1):
        w = lax.axis_index("s") * _NC + lax.axis_index("c")
        pltpu.sync_copy(tok_hbm.at[:, w, :, :], idx_v)
        pltpu.sync_copy(pos_hbm, pos_v)

        lane = lax.iota(jnp.int32, _L)
        row_ids = [lane + bg * _L for bg in range(NG)]

        def idx_row(t):
            return idx_v.at[lax.shift_right_logical(t, 3), lax.bitwise_and(t, 7)]

        def start_gather(t, gbuf, gsem):
            pltpu.async_copy(tab_hbm.at[idx_row(t)], gbuf, gsem)

        def wait_gather(t, gbuf, gsem):
            pltpu.make_async_copy(tab_hbm.at[idx_row(t)], gbuf, gsem).wait()

        def start_write(t, obuf, wsem):
            pltpu.async_copy(obuf, out_hbm.at[t, :, w, :, :], wsem)

        def wait_write(t, obuf, wsem):
            pltpu.make_async_copy(obuf, out_hbm.at[t, :, w, :, :], wsem).wait()

        def transpose_add(t, gbuf, obuf):
            t_splat = jnp.full((_L,), 0, dtype=jnp.int32) + t

            @plsc.parallel_loop(0, _L, unroll=2)
            def _diag_s(sft):
                rot = lax.bitwise_and(lane + sft, _L - 1)
                for q in range(D // _L):
                    col_ids = rot + q * _L
                    p = plsc.load_gather(pos_v, [t_splat, col_ids])
                    cb_ids = lax.shift_right_logical(col_ids, 3)
                    ci_ids = lax.bitwise_and(col_ids, 7)
                    for bg in range(NG):
                        v = plsc.load_gather(gbuf, [row_ids[bg], col_ids])
                        plsc.store_scatter(
                            obuf, [cb_ids, ci_ids, row_ids[bg]], v + p)

        start_gather(0, gbuf0, g_sem0)
        start_gather(1, gbuf1, g_sem1)

        def pair_body(k, carry):
            t0 = 2 * k
            t1 = t0 + 1

            wait_gather(t0, gbuf0, g_sem0)

            @pl.when(k > 0)
            def _():
                wait_write(t0 - 2, obuf0, w_sem0)

            transpose_add(t0, gbuf0, obuf0)
            start_write(t0, obuf0, w_sem0)

            @pl.when(k < KP - 1)
            def _():
                start_gather(t0 + 2, gbuf0, g_sem0)

            wait_gather(t1, gbuf1, g_sem1)

            @pl.when(k > 0)
            def _():
                wait_write(t1 - 2, obuf1, w_sem1)

            transpose_add(t1, gbuf1, obuf1)
            start_write(t1, obuf1, w_sem1)

            @pl.when(k < KP - 1)
            def _():
                start_gather(t1 + 2, gbuf1, g_sem1)

            return carry

        lax.fori_loop(0, KP, pair_body, 0)

        wait_write(T - 2, obuf0, w_sem0)
        wait_write(T - 1, obuf1, w_sem1)

    return emb_kernel(tokens4, token_table, position_embedding)


def kernel(tokens, token_table, position_embedding):
    B, T = tokens.shape
    D = token_table.shape[1]
    tp = T + (-T) % 8
    tokens_p = jnp.pad(tokens.T.astype(jnp.int32), ((0, tp - T), (0, 0)))
    tokens4 = (tokens_p.reshape(tp // 8, 8, B // 128, 128)
               .transpose(0, 2, 1, 3))
    out5 = _embedding_lookup(tokens4, token_table, position_embedding, T)
    return out5.transpose(2, 4, 0, 1, 3).reshape(B, T, D)
